# baseline (speedup 1.0000x reference)
import functools

import jax
import jax.numpy as jnp
from jax import lax
from jax.experimental import pallas as pl
from jax.experimental.pallas import tpu as pltpu
from jax.experimental.pallas import tpu_sc as plsc

_NC = 2
_NS = 16
_NW = _NC * _NS

_NB = 2
_CH = 256


@jax.jit
def _sc_gather(table, ids):
    n_chunks_total, ch = ids.shape
    n = n_chunks_total * ch
    d = table.shape[1]
    per = n // _NW
    n_chunks = per // ch
    n_blocks = n_chunks // _NB
    mesh = plsc.VectorSubcoreMesh(core_axis_name="c", subcore_axis_name="s")

    @functools.partial(
        pl.kernel,
        out_type=jax.ShapeDtypeStruct((n, d), jnp.float32),
        mesh=mesh,
        scratch_types=[
            pltpu.VMEM((n_chunks, ch), jnp.int32),
            pltpu.VMEM((_NB, ch, d), jnp.float32),
            pltpu.SemaphoreType.DMA((_NB,)),
            pltpu.SemaphoreType.DMA((_NB,)),
        ],
        compiler_params=pltpu.CompilerParams(use_tc_tiling_on_sc=False),
    )
    def k(table_hbm, ids_hbm, out_hbm, idx_v, rows_v, sem_g, sem_w):
        wid = lax.axis_index("s") * _NC + lax.axis_index("c")
        row0 = wid * per

        pltpu.sync_copy(ids_hbm.at[pl.ds(wid * n_chunks, n_chunks)], idx_v)

        def start_gather(c, slot):
            pltpu.async_copy(table_hbm.at[idx_v.at[c]], rows_v.at[slot],
                             sem_g.at[slot])

        def wait_gather(c, slot):
            pltpu.make_async_copy(table_hbm.at[idx_v.at[c]], rows_v.at[slot],
                                  sem_g.at[slot]).wait()

        def start_write(c, slot):
            pltpu.async_copy(rows_v.at[slot],
                             out_hbm.at[pl.ds(row0 + c * ch, ch)],
                             sem_w.at[slot])

        def wait_write(c, slot):
            pltpu.make_async_copy(rows_v.at[slot],
                                  out_hbm.at[pl.ds(row0 + c * ch, ch)],
                                  sem_w.at[slot]).wait()

        for s in range(_NB):
            start_gather(s, s)
        for s in range(_NB):
            wait_gather(s, s)
            start_write(s, s)

        def block_body(bidx, carry):
            c0 = bidx * _NB
            for s in range(_NB):
                wait_write(c0 - _NB + s, s)
                start_gather(c0 + s, s)
            for s in range(_NB):
                wait_gather(c0 + s, s)
                start_write(c0 + s, s)
            return carry

        lax.fori_loop(1, n_blocks, block_body, 0)

        c0 = (n_blocks - 1) * _NB
        for s in range(_NB):
            wait_write(c0 + s, s)

    return k(table, ids)


def kernel(token_ids, W):
    t, p = token_ids.shape
    ids = token_ids.astype(jnp.int32).reshape(-1, _CH)
    out = _sc_gather(W, ids)
    return out.reshape(t, p, W.shape[1])


# speedup vs baseline: 1.0154x; 1.0154x over previous
import functools

import jax
import jax.numpy as jnp
from jax import lax
from jax.experimental import pallas as pl
from jax.experimental.pallas import tpu as pltpu
from jax.experimental.pallas import tpu_sc as plsc

_NC = 2
_NS = 16
_NW = _NC * _NS

_NB = 4
_CH = 256


@jax.jit
def _sc_gather(table, ids):
    n_chunks_total, ch = ids.shape
    n = n_chunks_total * ch
    d = table.shape[1]
    per = n // _NW
    n_chunks = per // ch
    n_blocks = n_chunks // _NB
    mesh = plsc.VectorSubcoreMesh(core_axis_name="c", subcore_axis_name="s")

    @functools.partial(
        pl.kernel,
        out_type=jax.ShapeDtypeStruct((n, d), jnp.float32),
        mesh=mesh,
        scratch_types=[
            pltpu.VMEM((n_chunks, ch), jnp.int32),
            pltpu.VMEM((_NB, ch, d), jnp.float32),
            pltpu.SemaphoreType.DMA((_NB,)),
            pltpu.SemaphoreType.DMA((_NB,)),
        ],
        compiler_params=pltpu.CompilerParams(use_tc_tiling_on_sc=False),
    )
    def k(table_hbm, ids_hbm, out_hbm, idx_v, rows_v, sem_g, sem_w):
        wid = lax.axis_index("s") * _NC + lax.axis_index("c")
        row0 = wid * per

        pltpu.sync_copy(ids_hbm.at[pl.ds(wid * n_chunks, n_chunks)], idx_v)

        def start_gather(c, slot):
            pltpu.async_copy(table_hbm.at[idx_v.at[c]], rows_v.at[slot],
                             sem_g.at[slot])

        def wait_gather(c, slot):
            pltpu.make_async_copy(table_hbm.at[idx_v.at[c]], rows_v.at[slot],
                                  sem_g.at[slot]).wait()

        def start_write(c, slot):
            pltpu.async_copy(rows_v.at[slot],
                             out_hbm.at[pl.ds(row0 + c * ch, ch)],
                             sem_w.at[slot])

        def wait_write(c, slot):
            pltpu.make_async_copy(rows_v.at[slot],
                                  out_hbm.at[pl.ds(row0 + c * ch, ch)],
                                  sem_w.at[slot]).wait()

        for s in range(_NB):
            start_gather(s, s)
        for s in range(_NB):
            wait_gather(s, s)
            start_write(s, s)

        def block_body(bidx, carry):
            c0 = bidx * _NB
            for s in range(_NB):
                wait_write(c0 - _NB + s, s)
                start_gather(c0 + s, s)
            for s in range(_NB):
                wait_gather(c0 + s, s)
                start_write(c0 + s, s)
            return carry

        lax.fori_loop(1, n_blocks, block_body, 0)

        c0 = (n_blocks - 1) * _NB
        for s in range(_NB):
            wait_write(c0 + s, s)

    return k(table, ids)


def kernel(token_ids, W):
    t, p = token_ids.shape
    ids = token_ids.astype(jnp.int32).reshape(-1, _CH)
    out = _sc_gather(W, ids)
    return out.reshape(t, p, W.shape[1])
